# baseline (speedup 1.0000x reference)
import functools

import jax
import jax.numpy as jnp
from jax import lax
from jax.experimental import pallas as pl
from jax.experimental.pallas import tpu as pltpu
from jax.experimental.pallas import tpu_sc as plsc

_NUM_FIELDS = 26
_FIELD_SIZE = 100000
_D = 16
_B = 16384
_NW = 32
_BW = _B // _NW
_C = 64
_NCHUNK = _BW // _C
_IPC = _C * _NUM_FIELDS
_KSUB = _IPC // 128

_R = _NUM_FIELDS * _FIELD_SIZE
_FULL_BLOCKS = _R // 128
_TAIL = _R - _FULL_BLOCKS * 128


def _transpose_kernel(tt_hbm, out_hbm, buf, outb, colv, buf64, outb64):
    cid = lax.axis_index("c")
    sid = lax.axis_index("s")
    wid = sid * 2 + cid
    iota = lax.iota(jnp.int32, _D)

    nblk = _FULL_BLOCKS // _NW + jnp.where(wid < _FULL_BLOCKS % _NW, 1, 0)

    def blk_body(i, carry):
        blk = wid + i * _NW
        pltpu.sync_copy(tt_hbm.at[:, pl.ds(blk * 128, 128)], buf)

        def col_body(j, carry2):
            colv[...] = jnp.full((_D,), j, jnp.int32)
            outb[pl.ds(j * _D, _D)] = plsc.load_gather(buf, [iota, colv[...]])
            return carry2

        lax.fori_loop(0, 128, col_body, 0, unroll=4)
        pltpu.sync_copy(outb, out_hbm.at[pl.ds(blk * (128 * _D), 128 * _D)])
        return carry

    lax.fori_loop(0, nblk, blk_body, 0)

    @pl.when(wid == _NW - 1)
    def _():
        pltpu.sync_copy(tt_hbm.at[:, pl.ds(_FULL_BLOCKS * 128, _TAIL)], buf64)

        def tail_body(j, carry2):
            colv[...] = jnp.full((_D,), j, jnp.int32)
            outb64[pl.ds(j * _D, _D)] = plsc.load_gather(buf64, [iota, colv[...]])
            return carry2

        lax.fori_loop(0, _TAIL, tail_body, 0, unroll=4)
        pltpu.sync_copy(
            outb64, out_hbm.at[pl.ds(_FULL_BLOCKS * (128 * _D), _TAIL * _D)]
        )


def _fm_gather_kernel(xf_hbm, table_hbm, out_hbm, idx_v, pat_v, rows_v, outb, sem):
    cid = lax.axis_index("c")
    sid = lax.axis_index("s")
    wid = sid * 2 + cid
    iota = lax.iota(jnp.int32, 16)

    for j in range(_IPC // 16):
        sl = pl.ds(j * 16, 16)
        pat_v[sl] = ((iota + j * 16) % _NUM_FIELDS) * _FIELD_SIZE

    def chunk_body(g, carry):
        base = wid * (_NCHUNK * _IPC) + g * _IPC
        pltpu.sync_copy(xf_hbm.at[pl.ds(base, _IPC)], idx_v)
        for j in range(_IPC // 16):
            sl = pl.ds(j * 16, 16)
            idx_v[sl] = idx_v[sl] + pat_v[sl]
        copies = [
            pltpu.async_copy(
                table_hbm.at[idx_v.at[pl.ds(k * 128, 128)]],
                rows_v.at[pl.ds(k * 128, 128)],
                sem,
            )
            for k in range(_KSUB)
        ]
        for cpy in copies:
            cpy.wait()

        def row_fn(i, carry2):
            b0 = i * _NUM_FIELDS
            e = rows_v[b0]
            s = e
            q = e * e
            for f in range(1, _NUM_FIELDS):
                e = rows_v[b0 + f]
                s = s + e
                q = q + e * e
            outb[pl.ds(i * 16, 16)] = s * s - q
            return carry2

        lax.fori_loop(0, _C, row_fn, 0, unroll=2)
        pltpu.sync_copy(outb, out_hbm.at[pl.ds((wid * _BW + g * _C) * _D, _C * _D)])
        return carry

    lax.fori_loop(0, _NCHUNK, chunk_body, 0)


def _rowsum_kernel(r_ref, o_ref):
    m = (lax.broadcasted_iota(jnp.int32, (128, 8), 0) // _D
         == lax.broadcasted_iota(jnp.int32, (128, 8), 1)).astype(jnp.float32)
    o_ref[...] = 0.5 * jnp.dot(r_ref[...], m, preferred_element_type=jnp.float32)


def kernel(x, table):
    mesh = plsc.VectorSubcoreMesh(core_axis_name="c", subcore_axis_name="s")

    tt = jnp.swapaxes(table, 0, 1)
    transpose = functools.partial(
        pl.kernel,
        mesh=mesh,
        out_type=jax.ShapeDtypeStruct((_R * _D,), jnp.float32),
        scratch_types=[
            pltpu.VMEM((_D, 128), jnp.float32),
            pltpu.VMEM((128 * _D,), jnp.float32),
            pltpu.VMEM((_D,), jnp.int32),
            pltpu.VMEM((_D, _TAIL), jnp.float32),
            pltpu.VMEM((_TAIL * _D,), jnp.float32),
        ],
        compiler_params=pltpu.CompilerParams(
            use_tc_tiling_on_sc=True, needs_layout_passes=False
        ),
    )(_transpose_kernel)
    table_rm = transpose(tt).reshape(_R, _D)

    xf = x.astype(jnp.int32).reshape(_B * _NUM_FIELDS)
    fm = functools.partial(
        pl.kernel,
        mesh=mesh,
        out_type=jax.ShapeDtypeStruct((_B * _D,), jnp.float32),
        scratch_types=[
            pltpu.VMEM((_IPC,), jnp.int32),
            pltpu.VMEM((_IPC,), jnp.int32),
            pltpu.VMEM((_IPC, _D), jnp.float32),
            pltpu.VMEM((_C * _D,), jnp.float32),
            pltpu.SemaphoreType.DMA,
        ],
        compiler_params=pltpu.CompilerParams(use_tc_tiling_on_sc=False),
    )(_fm_gather_kernel)
    partial = fm(xf, table_rm).reshape(_B * _D // 128, 128)
    out = pl.pallas_call(
        _rowsum_kernel,
        out_shape=jax.ShapeDtypeStruct((_B * _D // 128, 8), jnp.float32),
    )(partial)
    return out.reshape(_B)


# speedup vs baseline: 1.5025x; 1.5025x over previous
import functools

import jax
import jax.numpy as jnp
from jax import lax
from jax.experimental import pallas as pl
from jax.experimental.pallas import tpu as pltpu
from jax.experimental.pallas import tpu_sc as plsc

_NUM_FIELDS = 26
_FIELD_SIZE = 100000
_D = 16
_B = 16384
_NW = 32
_BW = _B // _NW
_C = 64
_NCHUNK = _BW // _C
_IPC = _C * _NUM_FIELDS
_KSUB = _IPC // 128

_R = _NUM_FIELDS * _FIELD_SIZE
_W = 1024
_NBLK = _R // _W
_TAIL = _R - _NBLK * _W
_WPW = -(-_NBLK // _NW)


def _transpose_kernel(
    tt_hbm, out_hbm, buf0, buf1, outb0, outb1, buf64, outb64, rsem, wsem
):
    cid = lax.axis_index("c")
    sid = lax.axis_index("s")
    wid = sid * 2 + cid
    iota = lax.iota(jnp.int32, _D)
    bufs = (buf0, buf1)
    outbs = (outb0, outb1)

    def blk_of(i):
        return jnp.minimum(wid * _WPW + i, _NBLK - 1)

    def read(i):
        return pltpu.async_copy(
            tt_hbm.at[:, pl.ds(blk_of(i) * _W, _W)], bufs[i % 2], rsem
        )

    def transpose_block(buf, outb):
        def col_body(j, col):
            outb[pl.ds(j * _D, _D)] = plsc.load_gather(buf, [iota, col])
            return col + 1

        lax.fori_loop(0, _W, col_body, jnp.zeros((_D,), jnp.int32), unroll=8)

    writes = [None, None]
    rd = [read(0), read(1)]
    for i in range(_WPW):
        rd[i % 2].wait()
        if writes[i % 2] is not None:
            writes[i % 2].wait()
        transpose_block(bufs[i % 2], outbs[i % 2])
        writes[i % 2] = pltpu.async_copy(
            outbs[i % 2], out_hbm.at[pl.ds(blk_of(i) * (_W * _D), _W * _D)], wsem
        )
        if i + 2 < _WPW:
            rd[i % 2] = read(i + 2)
    for w in writes:
        if w is not None:
            w.wait()

    @pl.when(wid == _NW - 1)
    def _():
        pltpu.sync_copy(tt_hbm.at[:, pl.ds(_NBLK * _W, _TAIL)], buf64)

        def tail_body(j, col):
            outb64[pl.ds(j * _D, _D)] = plsc.load_gather(buf64, [iota, col])
            return col + 1

        lax.fori_loop(0, _TAIL, tail_body, jnp.zeros((_D,), jnp.int32), unroll=4)
        pltpu.sync_copy(outb64, out_hbm.at[pl.ds(_NBLK * (_W * _D), _TAIL * _D)])


def _fm_gather_kernel(xf_hbm, table_hbm, out_hbm, idx_v, pat_v, rows_v, outb, sem):
    cid = lax.axis_index("c")
    sid = lax.axis_index("s")
    wid = sid * 2 + cid
    iota = lax.iota(jnp.int32, 16)

    for j in range(_IPC // 16):
        sl = pl.ds(j * 16, 16)
        pat_v[sl] = ((iota + j * 16) % _NUM_FIELDS) * _FIELD_SIZE

    def chunk_body(g, carry):
        base = wid * (_NCHUNK * _IPC) + g * _IPC
        pltpu.sync_copy(xf_hbm.at[pl.ds(base, _IPC)], idx_v)
        for j in range(_IPC // 16):
            sl = pl.ds(j * 16, 16)
            idx_v[sl] = idx_v[sl] + pat_v[sl]
        copies = [
            pltpu.async_copy(
                table_hbm.at[idx_v.at[pl.ds(k * 128, 128)]],
                rows_v.at[pl.ds(k * 128, 128)],
                sem,
            )
            for k in range(_KSUB)
        ]
        for cpy in copies:
            cpy.wait()

        def row_fn(i, carry2):
            b0 = i * _NUM_FIELDS
            e = rows_v[b0]
            s = e
            q = e * e
            for f in range(1, _NUM_FIELDS):
                e = rows_v[b0 + f]
                s = s + e
                q = q + e * e
            outb[pl.ds(i * 16, 16)] = s * s - q
            return carry2

        lax.fori_loop(0, _C, row_fn, 0, unroll=2)
        pltpu.sync_copy(outb, out_hbm.at[pl.ds((wid * _BW + g * _C) * _D, _C * _D)])
        return carry

    lax.fori_loop(0, _NCHUNK, chunk_body, 0)


def _rowsum_kernel(r_ref, o_ref):
    m = (lax.broadcasted_iota(jnp.int32, (128, 8), 0) // _D
         == lax.broadcasted_iota(jnp.int32, (128, 8), 1)).astype(jnp.float32)
    o_ref[...] = 0.5 * jnp.dot(r_ref[...], m, preferred_element_type=jnp.float32)


def kernel(x, table):
    mesh = plsc.VectorSubcoreMesh(core_axis_name="c", subcore_axis_name="s")

    tt = jnp.swapaxes(table, 0, 1)
    transpose = functools.partial(
        pl.kernel,
        mesh=mesh,
        out_type=jax.ShapeDtypeStruct((_R * _D,), jnp.float32),
        scratch_types=[
            pltpu.VMEM((_D, _W), jnp.float32),
            pltpu.VMEM((_D, _W), jnp.float32),
            pltpu.VMEM((_W * _D,), jnp.float32),
            pltpu.VMEM((_W * _D,), jnp.float32),
            pltpu.VMEM((_D, _TAIL), jnp.float32),
            pltpu.VMEM((_TAIL * _D,), jnp.float32),
            pltpu.SemaphoreType.DMA,
            pltpu.SemaphoreType.DMA,
        ],
        compiler_params=pltpu.CompilerParams(
            use_tc_tiling_on_sc=True, needs_layout_passes=False
        ),
    )(_transpose_kernel)
    table_rm = transpose(tt).reshape(_R, _D)

    xf = x.astype(jnp.int32).reshape(_B * _NUM_FIELDS)
    fm = functools.partial(
        pl.kernel,
        mesh=mesh,
        out_type=jax.ShapeDtypeStruct((_B * _D,), jnp.float32),
        scratch_types=[
            pltpu.VMEM((_IPC,), jnp.int32),
            pltpu.VMEM((_IPC,), jnp.int32),
            pltpu.VMEM((_IPC, _D), jnp.float32),
            pltpu.VMEM((_C * _D,), jnp.float32),
            pltpu.SemaphoreType.DMA,
        ],
        compiler_params=pltpu.CompilerParams(use_tc_tiling_on_sc=False),
    )(_fm_gather_kernel)
    partial = fm(xf, table_rm).reshape(_B * _D // 128, 128)
    out = pl.pallas_call(
        _rowsum_kernel,
        out_shape=jax.ShapeDtypeStruct((_B * _D // 128, 8), jnp.float32),
    )(partial)
    return out.reshape(_B)


# speedup vs baseline: 2.4100x; 1.6040x over previous
import functools

import jax
import jax.numpy as jnp
from jax import lax
from jax.experimental import pallas as pl
from jax.experimental.pallas import tpu as pltpu
from jax.experimental.pallas import tpu_sc as plsc

_NUM_FIELDS = 26
_FIELD_SIZE = 100000
_D = 16
_B = 16384
_NW = 32
_BW = _B // _NW
_C = 64
_NCHUNK = _BW // _C
_IPC = _C * _NUM_FIELDS
_KSUB = _IPC // 128

_R = _NUM_FIELDS * _FIELD_SIZE
_W = 1024
_NBLK = _R // _W
_TAIL = _R - _NBLK * _W
_WPW = -(-_NBLK // _NW)


def _transpose_kernel(
    tt_hbm, out_hbm, buf0, buf1, outb0, outb1, buf64, outb64, rsem, wsem
):
    cid = lax.axis_index("c")
    sid = lax.axis_index("s")
    wid = sid * 2 + cid
    iota = lax.iota(jnp.int32, _D)
    bufs = (buf0, buf1)
    outbs = (outb0, outb1)

    def blk_of(i):
        return jnp.minimum(wid * _WPW + i, _NBLK - 1)

    def read(i):
        return pltpu.async_copy(
            tt_hbm.at[:, pl.ds(blk_of(i) * _W, _W)], bufs[i % 2], rsem
        )

    def transpose_block(buf, outb):
        def col_body(j8, col):
            vals = [plsc.load_gather(buf, [iota, col + c]) for c in range(8)]
            for c in range(8):
                outb[pl.ds((j8 * 8 + c) * _D, _D)] = vals[c]
            return col + 8

        lax.fori_loop(0, _W // 8, col_body, jnp.zeros((_D,), jnp.int32))

    writes = [None, None]
    rd = [read(0), read(1)]
    for i in range(_WPW):
        rd[i % 2].wait()
        if writes[i % 2] is not None:
            writes[i % 2].wait()
        transpose_block(bufs[i % 2], outbs[i % 2])
        writes[i % 2] = pltpu.async_copy(
            outbs[i % 2], out_hbm.at[pl.ds(blk_of(i) * (_W * _D), _W * _D)], wsem
        )
        if i + 2 < _WPW:
            rd[i % 2] = read(i + 2)
    for w in writes:
        if w is not None:
            w.wait()

    @pl.when(wid == _NW - 1)
    def _():
        pltpu.sync_copy(tt_hbm.at[:, pl.ds(_NBLK * _W, _TAIL)], buf64)

        def tail_body(j, col):
            outb64[pl.ds(j * _D, _D)] = plsc.load_gather(buf64, [iota, col])
            return col + 1

        lax.fori_loop(0, _TAIL, tail_body, jnp.zeros((_D,), jnp.int32), unroll=4)
        pltpu.sync_copy(outb64, out_hbm.at[pl.ds(_NBLK * (_W * _D), _TAIL * _D)])


def _fm_gather_kernel(xf_hbm, table_hbm, out_hbm, idx_v, pat_v, rows_v, outb, sem):
    cid = lax.axis_index("c")
    sid = lax.axis_index("s")
    wid = sid * 2 + cid
    iota = lax.iota(jnp.int32, 16)

    for j in range(_IPC // 16):
        sl = pl.ds(j * 16, 16)
        pat_v[sl] = ((iota + j * 16) % _NUM_FIELDS) * _FIELD_SIZE

    def chunk_body(g, carry):
        base = wid * (_NCHUNK * _IPC) + g * _IPC
        pltpu.sync_copy(xf_hbm.at[pl.ds(base, _IPC)], idx_v)
        for j in range(_IPC // 16):
            sl = pl.ds(j * 16, 16)
            idx_v[sl] = idx_v[sl] + pat_v[sl]
        copies = [
            pltpu.async_copy(
                table_hbm.at[idx_v.at[pl.ds(k * 128, 128)]],
                rows_v.at[pl.ds(k * 128, 128)],
                sem,
            )
            for k in range(_KSUB)
        ]
        for cpy in copies:
            cpy.wait()

        def row_fn(i, carry2):
            b0 = i * _NUM_FIELDS
            e = rows_v[b0]
            s = e
            q = e * e
            for f in range(1, _NUM_FIELDS):
                e = rows_v[b0 + f]
                s = s + e
                q = q + e * e
            outb[pl.ds(i * 16, 16)] = s * s - q
            return carry2

        lax.fori_loop(0, _C, row_fn, 0, unroll=2)
        pltpu.sync_copy(outb, out_hbm.at[pl.ds((wid * _BW + g * _C) * _D, _C * _D)])
        return carry

    lax.fori_loop(0, _NCHUNK, chunk_body, 0)


def _rowsum_kernel(r_ref, o_ref):
    m = (lax.broadcasted_iota(jnp.int32, (128, 8), 0) // _D
         == lax.broadcasted_iota(jnp.int32, (128, 8), 1)).astype(jnp.float32)
    o_ref[...] = 0.5 * jnp.dot(r_ref[...], m, preferred_element_type=jnp.float32)


def kernel(x, table):
    mesh = plsc.VectorSubcoreMesh(core_axis_name="c", subcore_axis_name="s")

    tt = jnp.swapaxes(table, 0, 1)
    transpose = functools.partial(
        pl.kernel,
        mesh=mesh,
        out_type=jax.ShapeDtypeStruct((_R * _D,), jnp.float32),
        scratch_types=[
            pltpu.VMEM((_D, _W), jnp.float32),
            pltpu.VMEM((_D, _W), jnp.float32),
            pltpu.VMEM((_W * _D,), jnp.float32),
            pltpu.VMEM((_W * _D,), jnp.float32),
            pltpu.VMEM((_D, _TAIL), jnp.float32),
            pltpu.VMEM((_TAIL * _D,), jnp.float32),
            pltpu.SemaphoreType.DMA,
            pltpu.SemaphoreType.DMA,
        ],
        compiler_params=pltpu.CompilerParams(
            use_tc_tiling_on_sc=True, needs_layout_passes=False
        ),
    )(_transpose_kernel)
    table_rm = transpose(tt).reshape(_R, _D)

    xf = x.astype(jnp.int32).reshape(_B * _NUM_FIELDS)
    fm = functools.partial(
        pl.kernel,
        mesh=mesh,
        out_type=jax.ShapeDtypeStruct((_B * _D,), jnp.float32),
        scratch_types=[
            pltpu.VMEM((_IPC,), jnp.int32),
            pltpu.VMEM((_IPC,), jnp.int32),
            pltpu.VMEM((_IPC, _D), jnp.float32),
            pltpu.VMEM((_C * _D,), jnp.float32),
            pltpu.SemaphoreType.DMA,
        ],
        compiler_params=pltpu.CompilerParams(use_tc_tiling_on_sc=False),
    )(_fm_gather_kernel)
    partial = fm(xf, table_rm).reshape(_B * _D // 128, 128)
    out = pl.pallas_call(
        _rowsum_kernel,
        out_shape=jax.ShapeDtypeStruct((_B * _D // 128, 8), jnp.float32),
    )(partial)
    return out.reshape(_B)


# speedup vs baseline: 2.4823x; 1.0300x over previous
import functools

import jax
import jax.numpy as jnp
from jax import lax
from jax.experimental import pallas as pl
from jax.experimental.pallas import tpu as pltpu
from jax.experimental.pallas import tpu_sc as plsc

_NUM_FIELDS = 26
_FIELD_SIZE = 100000
_D = 16
_B = 16384
_NW = 32
_BW = _B // _NW
_C = 64
_NCHUNK = _BW // _C
_IPC = _C * _NUM_FIELDS
_KSUB = _IPC // 128

_R = _NUM_FIELDS * _FIELD_SIZE
_W = 1024
_NBLK = _R // _W
_TAIL = _R - _NBLK * _W
_WPW = -(-_NBLK // _NW)


def _transpose_kernel(
    tt_hbm, out_hbm, buf0, buf1, outb0, outb1, buf64, outb64, rsem, wsem
):
    cid = lax.axis_index("c")
    sid = lax.axis_index("s")
    wid = sid * 2 + cid
    iota = lax.iota(jnp.int32, _D)
    bufs = (buf0, buf1)
    outbs = (outb0, outb1)

    def blk_of(i):
        return jnp.minimum(wid * _WPW + i, _NBLK - 1)

    def read(i):
        return pltpu.async_copy(
            tt_hbm.at[:, pl.ds(blk_of(i) * _W, _W)], bufs[i % 2], rsem
        )

    def transpose_block(buf, outb):
        def col_body(j8, col):
            vals = [plsc.load_gather(buf, [iota, col + c]) for c in range(16)]
            for c in range(16):
                outb[pl.ds((j8 * 16 + c) * _D, _D)] = vals[c]
            return col + 16

        lax.fori_loop(0, _W // 16, col_body, jnp.zeros((_D,), jnp.int32))

    writes = [None, None]
    rd = [read(0), read(1)]
    for i in range(_WPW):
        rd[i % 2].wait()
        if writes[i % 2] is not None:
            writes[i % 2].wait()
        transpose_block(bufs[i % 2], outbs[i % 2])
        writes[i % 2] = pltpu.async_copy(
            outbs[i % 2], out_hbm.at[pl.ds(blk_of(i) * (_W * _D), _W * _D)], wsem
        )
        if i + 2 < _WPW:
            rd[i % 2] = read(i + 2)
    for w in writes:
        if w is not None:
            w.wait()

    @pl.when(wid == _NW - 1)
    def _():
        pltpu.sync_copy(tt_hbm.at[:, pl.ds(_NBLK * _W, _TAIL)], buf64)

        def tail_body(j, col):
            outb64[pl.ds(j * _D, _D)] = plsc.load_gather(buf64, [iota, col])
            return col + 1

        lax.fori_loop(0, _TAIL, tail_body, jnp.zeros((_D,), jnp.int32), unroll=4)
        pltpu.sync_copy(outb64, out_hbm.at[pl.ds(_NBLK * (_W * _D), _TAIL * _D)])


def _fm_gather_kernel(xf_hbm, table_hbm, out_hbm, idx_v, pat_v, rows_v, outb, sem):
    cid = lax.axis_index("c")
    sid = lax.axis_index("s")
    wid = sid * 2 + cid
    iota = lax.iota(jnp.int32, 16)

    for j in range(_IPC // 16):
        sl = pl.ds(j * 16, 16)
        pat_v[sl] = ((iota + j * 16) % _NUM_FIELDS) * _FIELD_SIZE

    def chunk_body(g, carry):
        base = wid * (_NCHUNK * _IPC) + g * _IPC
        pltpu.sync_copy(xf_hbm.at[pl.ds(base, _IPC)], idx_v)
        for j in range(_IPC // 16):
            sl = pl.ds(j * 16, 16)
            idx_v[sl] = idx_v[sl] + pat_v[sl]
        copies = [
            pltpu.async_copy(
                table_hbm.at[idx_v.at[pl.ds(k * 128, 128)]],
                rows_v.at[pl.ds(k * 128, 128)],
                sem,
            )
            for k in range(_KSUB)
        ]
        for cpy in copies:
            cpy.wait()

        def row_fn(i, carry2):
            b0 = i * _NUM_FIELDS
            e = rows_v[b0]
            s = e
            q = e * e
            for f in range(1, _NUM_FIELDS):
                e = rows_v[b0 + f]
                s = s + e
                q = q + e * e
            outb[pl.ds(i * 16, 16)] = s * s - q
            return carry2

        lax.fori_loop(0, _C, row_fn, 0, unroll=2)
        pltpu.sync_copy(outb, out_hbm.at[pl.ds((wid * _BW + g * _C) * _D, _C * _D)])
        return carry

    lax.fori_loop(0, _NCHUNK, chunk_body, 0)


def _rowsum_kernel(r_ref, o_ref):
    m = (lax.broadcasted_iota(jnp.int32, (128, 8), 0) // _D
         == lax.broadcasted_iota(jnp.int32, (128, 8), 1)).astype(jnp.float32)
    o_ref[...] = 0.5 * jnp.dot(r_ref[...], m, preferred_element_type=jnp.float32)


def kernel(x, table):
    mesh = plsc.VectorSubcoreMesh(core_axis_name="c", subcore_axis_name="s")

    tt = jnp.swapaxes(table, 0, 1)
    transpose = functools.partial(
        pl.kernel,
        mesh=mesh,
        out_type=jax.ShapeDtypeStruct((_R * _D,), jnp.float32),
        scratch_types=[
            pltpu.VMEM((_D, _W), jnp.float32),
            pltpu.VMEM((_D, _W), jnp.float32),
            pltpu.VMEM((_W * _D,), jnp.float32),
            pltpu.VMEM((_W * _D,), jnp.float32),
            pltpu.VMEM((_D, _TAIL), jnp.float32),
            pltpu.VMEM((_TAIL * _D,), jnp.float32),
            pltpu.SemaphoreType.DMA,
            pltpu.SemaphoreType.DMA,
        ],
        compiler_params=pltpu.CompilerParams(
            use_tc_tiling_on_sc=True, needs_layout_passes=False
        ),
    )(_transpose_kernel)
    table_rm = transpose(tt).reshape(_R, _D)

    xf = x.astype(jnp.int32).reshape(_B * _NUM_FIELDS)
    fm = functools.partial(
        pl.kernel,
        mesh=mesh,
        out_type=jax.ShapeDtypeStruct((_B * _D,), jnp.float32),
        scratch_types=[
            pltpu.VMEM((_IPC,), jnp.int32),
            pltpu.VMEM((_IPC,), jnp.int32),
            pltpu.VMEM((_IPC, _D), jnp.float32),
            pltpu.VMEM((_C * _D,), jnp.float32),
            pltpu.SemaphoreType.DMA,
        ],
        compiler_params=pltpu.CompilerParams(use_tc_tiling_on_sc=False),
    )(_fm_gather_kernel)
    partial = fm(xf, table_rm).reshape(_B * _D // 128, 128)
    out = pl.pallas_call(
        _rowsum_kernel,
        out_shape=jax.ShapeDtypeStruct((_B * _D // 128, 8), jnp.float32),
    )(partial)
    return out.reshape(_B)


# speedup vs baseline: 3.7135x; 1.4960x over previous
import functools

import jax
import jax.numpy as jnp
from jax import lax
from jax.experimental import pallas as pl
from jax.experimental.pallas import tpu as pltpu
from jax.experimental.pallas import tpu_sc as plsc

_NUM_FIELDS = 26
_FIELD_SIZE = 100000
_D = 16
_B = 16384
_NW = 32
_BW = _B // _NW
_C = 64
_NCHUNK = _BW // _C
_IPC = _C * _NUM_FIELDS
_KSUB = _IPC // 128

_R = _NUM_FIELDS * _FIELD_SIZE
_W = 1024
_NBLK = _R // _W
_TAIL = _R - _NBLK * _W
_WPW = -(-_NBLK // _NW)


def _transpose_kernel(
    tt_hbm, out_hbm, buf0, buf1, outb0, outb1, buf64, outb64, rsem, wsem
):
    cid = lax.axis_index("c")
    sid = lax.axis_index("s")
    wid = sid * 2 + cid
    iota = lax.iota(jnp.int32, _D)
    bufs = (buf0, buf1)
    outbs = (outb0, outb1)

    def blk_of(i):
        return jnp.minimum(wid * _WPW + i, _NBLK - 1)

    def read(i):
        return pltpu.async_copy(
            tt_hbm.at[:, pl.ds(blk_of(i) * _W, _W)], bufs[i % 2], rsem
        )

    def transpose_block(buf, outb):
        def grp_body(k, base):
            def d_body(d, carry):
                plsc.store_scatter(outb, [base + d], buf[d, pl.ds(k * _D, _D)])
                return carry

            lax.fori_loop(0, _D, d_body, 0, unroll=4)
            return base + _D * _D

        lax.fori_loop(0, _W // _D, grp_body, iota * _D)

    writes = [None, None]
    rd = [read(0), read(1)]
    for i in range(_WPW):
        rd[i % 2].wait()
        if writes[i % 2] is not None:
            writes[i % 2].wait()
        transpose_block(bufs[i % 2], outbs[i % 2])
        writes[i % 2] = pltpu.async_copy(
            outbs[i % 2], out_hbm.at[pl.ds(blk_of(i) * (_W * _D), _W * _D)], wsem
        )
        if i + 2 < _WPW:
            rd[i % 2] = read(i + 2)
    for w in writes:
        if w is not None:
            w.wait()

    @pl.when(wid == _NW - 1)
    def _():
        pltpu.sync_copy(tt_hbm.at[:, pl.ds(_NBLK * _W, _TAIL)], buf64)

        def tail_body(j, col):
            outb64[pl.ds(j * _D, _D)] = plsc.load_gather(buf64, [iota, col])
            return col + 1

        lax.fori_loop(0, _TAIL, tail_body, jnp.zeros((_D,), jnp.int32), unroll=4)
        pltpu.sync_copy(outb64, out_hbm.at[pl.ds(_NBLK * (_W * _D), _TAIL * _D)])


def _fm_gather_kernel(xf_hbm, table_hbm, out_hbm, idx_v, pat_v, rows_v, outb, sem):
    cid = lax.axis_index("c")
    sid = lax.axis_index("s")
    wid = sid * 2 + cid
    iota = lax.iota(jnp.int32, 16)

    for j in range(_IPC // 16):
        sl = pl.ds(j * 16, 16)
        pat_v[sl] = ((iota + j * 16) % _NUM_FIELDS) * _FIELD_SIZE

    def chunk_body(g, carry):
        base = wid * (_NCHUNK * _IPC) + g * _IPC
        pltpu.sync_copy(xf_hbm.at[pl.ds(base, _IPC)], idx_v)
        for j in range(_IPC // 16):
            sl = pl.ds(j * 16, 16)
            idx_v[sl] = idx_v[sl] + pat_v[sl]
        copies = [
            pltpu.async_copy(
                table_hbm.at[idx_v.at[pl.ds(k * 128, 128)]],
                rows_v.at[pl.ds(k * 128, 128)],
                sem,
            )
            for k in range(_KSUB)
        ]
        for cpy in copies:
            cpy.wait()

        def row_fn(i, carry2):
            b0 = i * _NUM_FIELDS
            e = rows_v[b0]
            s = e
            q = e * e
            for f in range(1, _NUM_FIELDS):
                e = rows_v[b0 + f]
                s = s + e
                q = q + e * e
            outb[pl.ds(i * 16, 16)] = s * s - q
            return carry2

        lax.fori_loop(0, _C, row_fn, 0, unroll=2)
        pltpu.sync_copy(outb, out_hbm.at[pl.ds((wid * _BW + g * _C) * _D, _C * _D)])
        return carry

    lax.fori_loop(0, _NCHUNK, chunk_body, 0)


def _rowsum_kernel(r_ref, o_ref):
    m = (lax.broadcasted_iota(jnp.int32, (128, 8), 0) // _D
         == lax.broadcasted_iota(jnp.int32, (128, 8), 1)).astype(jnp.float32)
    o_ref[...] = 0.5 * jnp.dot(r_ref[...], m, preferred_element_type=jnp.float32)


def kernel(x, table):
    mesh = plsc.VectorSubcoreMesh(core_axis_name="c", subcore_axis_name="s")

    tt = jnp.swapaxes(table, 0, 1)
    transpose = functools.partial(
        pl.kernel,
        mesh=mesh,
        out_type=jax.ShapeDtypeStruct((_R * _D,), jnp.float32),
        scratch_types=[
            pltpu.VMEM((_D, _W), jnp.float32),
            pltpu.VMEM((_D, _W), jnp.float32),
            pltpu.VMEM((_W * _D,), jnp.float32),
            pltpu.VMEM((_W * _D,), jnp.float32),
            pltpu.VMEM((_D, _TAIL), jnp.float32),
            pltpu.VMEM((_TAIL * _D,), jnp.float32),
            pltpu.SemaphoreType.DMA,
            pltpu.SemaphoreType.DMA,
        ],
        compiler_params=pltpu.CompilerParams(
            use_tc_tiling_on_sc=True, needs_layout_passes=False
        ),
    )(_transpose_kernel)
    table_rm = transpose(tt).reshape(_R, _D)

    xf = x.astype(jnp.int32).reshape(_B * _NUM_FIELDS)
    fm = functools.partial(
        pl.kernel,
        mesh=mesh,
        out_type=jax.ShapeDtypeStruct((_B * _D,), jnp.float32),
        scratch_types=[
            pltpu.VMEM((_IPC,), jnp.int32),
            pltpu.VMEM((_IPC,), jnp.int32),
            pltpu.VMEM((_IPC, _D), jnp.float32),
            pltpu.VMEM((_C * _D,), jnp.float32),
            pltpu.SemaphoreType.DMA,
        ],
        compiler_params=pltpu.CompilerParams(use_tc_tiling_on_sc=False),
    )(_fm_gather_kernel)
    partial = fm(xf, table_rm).reshape(_B * _D // 128, 128)
    out = pl.pallas_call(
        _rowsum_kernel,
        out_shape=jax.ShapeDtypeStruct((_B * _D // 128, 8), jnp.float32),
    )(partial)
    return out.reshape(_B)
